# Initial kernel scaffold; baseline (speedup 1.0000x reference)
import jax
import jax.numpy as jnp
from jax.experimental import pallas as pl


def kernel(x, edge_index, W0, b0, W1, b1, W_out, b_out):
    raise NotImplementedError("write your pallas kernel here")


# speedup vs baseline: 10.5065x; 10.5065x over previous
import functools

import jax
import jax.numpy as jnp
from jax import lax
from jax.experimental import pallas as pl
from jax.experimental.pallas import tpu as pltpu
from jax.experimental.pallas import tpu_sc as plsc

N = 10000
E = 320000
D = 128
CHUNK = 128
NCHUNK = E // CHUNK
NC, NS = 2, 16
CH_PER_CORE = NCHUNK // NC
CH_FLOOR = CH_PER_CORE // NS
CH_REM = CH_PER_CORE % NS
NPAD = 10240
DPT = NPAD // NS

BN = 1000
GRID = N // BN

_MESH = plsc.VectorSubcoreMesh(
    core_axis_name="c", subcore_axis_name="s", num_cores=NC, num_subcores=NS)


def _zero_1d(ref, n):
    for j in range(n // 16):
        ref[pl.ds(16 * j, 16)] = jnp.zeros((16,), jnp.float32)


def _deg_body(ei_hbm, out_hbm, eib, ones_v, zbuf, acc):
    c = lax.axis_index("c")
    s = lax.axis_index("s")
    for j in range(8):
        ones_v[pl.ds(16 * j, 16)] = jnp.ones((16,), jnp.float32)
    _zero_1d(zbuf, DPT)
    pltpu.sync_copy(zbuf, acc.at[pl.ds(s * DPT, DPT)])
    plsc.subcore_barrier()
    nj = jnp.where(s < CH_REM, CH_FLOOR + 1, CH_FLOOR)

    def body(j, carry):
        ci = c * CH_PER_CORE + s + NS * j
        pltpu.sync_copy(ei_hbm.at[ci], eib)
        pltpu.sync_copy(ones_v, acc.at[eib.at[1]], add=True)
        return carry

    lax.fori_loop(0, nj, body, 0)
    plsc.subcore_barrier()
    pltpu.sync_copy(acc.at[pl.ds(s * DPT, DPT)],
                    out_hbm.at[c, pl.ds(s * DPT, DPT)])


_deg_call = pl.kernel(
    _deg_body,
    out_type=jax.ShapeDtypeStruct((NC, NPAD), jnp.float32),
    mesh=_MESH,
    scratch_types=[
        pltpu.VMEM((2, CHUNK), jnp.int32),
        pltpu.VMEM((CHUNK,), jnp.float32),
        pltpu.VMEM((DPT,), jnp.float32),
        pltpu.VMEM_SHARED((NPAD,), jnp.float32),
    ],
)


def _make_agg(ntab):
    def body(*refs):
        gs = refs[:ntab]
        ei_hbm = refs[ntab]
        out_hbm = refs[ntab + 1]
        eib, rows, zbuf, sem, acc = refs[ntab + 2:]
        c = lax.axis_index("c")
        s = lax.axis_index("s")

        def zrow(i, carry):
            for j in range(8):
                zbuf[i, pl.ds(16 * j, 16)] = jnp.zeros((16,), jnp.float32)
            return carry

        lax.fori_loop(0, CHUNK, zrow, 0)
        nj = jnp.where(s < CH_REM, CH_FLOOR + 1, CH_FLOOR)

        for t in range(ntab):
            for k in range(DPT // CHUNK):
                pltpu.sync_copy(zbuf, acc.at[pl.ds(s * DPT + k * CHUNK, CHUNK)])
            plsc.subcore_barrier()
            g = gs[t]

            def body_j(j, carry, g=g):
                ci = c * CH_PER_CORE + s + NS * j
                pltpu.sync_copy(ei_hbm.at[ci], eib)
                pltpu.async_copy(g.at[eib.at[0]], rows, sem).wait()
                pltpu.sync_copy(rows, acc.at[eib.at[1]], add=True)
                return carry

            lax.fori_loop(0, nj, body_j, 0)
            plsc.subcore_barrier()
            pltpu.sync_copy(acc.at[pl.ds(s * DPT, DPT)],
                            out_hbm.at[t, c, pl.ds(s * DPT, DPT)])
            plsc.subcore_barrier()

    return pl.kernel(
        body,
        out_type=jax.ShapeDtypeStruct((ntab, NC, NPAD, D), jnp.float32),
        mesh=_MESH,
        scratch_types=[
            pltpu.VMEM((2, CHUNK), jnp.int32),
            pltpu.VMEM((CHUNK, D), jnp.float32),
            pltpu.VMEM((CHUNK, D), jnp.float32),
            pltpu.SemaphoreType.DMA,
            pltpu.VMEM_SHARED((NPAD, D), jnp.float32),
        ],
    )


_agg1 = _make_agg(1)
_agg4 = _make_agg(4)

_DOT = dict(preferred_element_type=jnp.float32, precision=lax.Precision.HIGHEST)


def _tc1_body(deg_ref, x_ref, w_ref, b_ref, inv_ref, g1_ref):
    dsum = deg_ref[0] + deg_ref[1] + 1.0
    iv = lax.rsqrt(dsum)
    inv_ref[...] = iv
    hl = jnp.dot(x_ref[...], w_ref[...], **_DOT) + b_ref[...]
    g1_ref[...] = hl * iv


_tc1 = pl.pallas_call(
    _tc1_body,
    grid=(GRID,),
    in_specs=[
        pl.BlockSpec((NC, BN, 1), lambda i: (0, i, 0)),
        pl.BlockSpec((BN, D), lambda i: (i, 0)),
        pl.BlockSpec((D, D), lambda i: (0, 0)),
        pl.BlockSpec((1, D), lambda i: (0, 0)),
    ],
    out_specs=[
        pl.BlockSpec((BN, 1), lambda i: (i, 0)),
        pl.BlockSpec((BN, D), lambda i: (i, 0)),
    ],
    out_shape=[
        jax.ShapeDtypeStruct((N, 1), jnp.float32),
        jax.ShapeDtypeStruct((N, D), jnp.float32),
    ],
)


def _tc2_body(p_ref, g1_ref, inv_ref, w_ref, b_ref, g2_ref):
    iv = inv_ref[...]
    cur = iv * (p_ref[0] + p_ref[1] + g1_ref[...])
    v = jnp.zeros_like(cur)
    for t in range(4):
        v = v + (cur - v) * 0.5
        sp = (v >= 1.0).astype(jnp.float32)
        v = v - sp
        hl = jnp.dot(sp, w_ref[...], **_DOT) + b_ref[...]
        g2_ref[t] = hl * iv


_tc2 = pl.pallas_call(
    _tc2_body,
    grid=(GRID,),
    in_specs=[
        pl.BlockSpec((NC, BN, D), lambda i: (0, i, 0)),
        pl.BlockSpec((BN, D), lambda i: (i, 0)),
        pl.BlockSpec((BN, 1), lambda i: (i, 0)),
        pl.BlockSpec((D, D), lambda i: (0, 0)),
        pl.BlockSpec((1, D), lambda i: (0, 0)),
    ],
    out_specs=pl.BlockSpec((4, BN, D), lambda i: (0, i, 0)),
    out_shape=jax.ShapeDtypeStruct((4, N, D), jnp.float32),
)


def _tc3_body(pp_ref, g2_ref, inv_ref, w_ref, b_ref, out_ref):
    iv = inv_ref[...]
    v = jnp.zeros((BN, D), jnp.float32)
    acc = jnp.zeros((BN, D), jnp.float32)
    for t in range(4):
        cur = iv * (pp_ref[t, 0] + pp_ref[t, 1] + g2_ref[t])
        v = v + (cur - v) * 0.5
        sp = (v >= 1.0).astype(jnp.float32)
        v = v - sp
        acc = acc + sp
    out_ref[...] = jnp.dot(acc * 0.25, w_ref[...], **_DOT) + b_ref[...]


_tc3 = pl.pallas_call(
    _tc3_body,
    grid=(GRID,),
    in_specs=[
        pl.BlockSpec((4, NC, BN, D), lambda i: (0, 0, i, 0)),
        pl.BlockSpec((4, BN, D), lambda i: (0, i, 0)),
        pl.BlockSpec((BN, 1), lambda i: (i, 0)),
        pl.BlockSpec((D, D), lambda i: (0, 0)),
        pl.BlockSpec((1, D), lambda i: (0, 0)),
    ],
    out_specs=pl.BlockSpec((BN, D), lambda i: (i, 0)),
    out_shape=jax.ShapeDtypeStruct((N, D), jnp.float32),
)


def kernel(x, edge_index, W0, b0, W1, b1, W_out, b_out):
    src = edge_index[0].reshape(NCHUNK, CHUNK)
    dst = edge_index[1].reshape(NCHUNK, CHUNK)
    ei = jnp.stack([src, dst], axis=1)
    degp = _deg_call(ei).reshape(NC, NPAD, 1)
    inv, g1 = _tc1(degp, x, W0, b0.reshape(1, D))
    p1 = _agg1(g1, ei).reshape(NC, NPAD, D)
    g2 = _tc2(p1, g1, inv, W1, b1.reshape(1, D))
    pp = _agg4(g2[0], g2[1], g2[2], g2[3], ei)
    return _tc3(pp, g2, inv, W_out, b_out.reshape(1, D))
